# baseline (speedup 1.0000x reference)
import functools

import jax
import jax.numpy as jnp
from jax import lax
from jax.experimental import pallas as pl
from jax.experimental.pallas import tpu as pltpu
from jax.experimental.pallas import tpu_sc as plsc

_NUM_OBJS = 151
_NUM_RELS = 51
_BATCH = 16384
_NROWS = _NUM_OBJS * _NUM_OBJS

_NC = 2
_NS = 16
_NW = _NC * _NS
_L = 16

_CPC = _BATCH // _L
_P2COLS = _NUM_RELS - _NW
_P2PW = _P2COLS * _CPC // _NW
_GRP = 32
_NGRP = _P2PW // _GRP


def _freq_bias_body(idx_hbm, tableT_hbm, outT_hbm,
                    idx_v, row0_v, row1_v, row2_v, col0_v, col2_v,
                    sem0, sem1, sem2, semw):
    wid = lax.axis_index("s") * _NC + lax.axis_index("c")

    q0 = wid * _P2PW
    c1 = _NW + q0 // _CPC
    r1 = q0 % _CPC
    n1 = jnp.minimum(_CPC - r1, _P2PW)
    has2 = n1 < _P2PW

    cp_idx = pltpu.async_copy(idx_hbm, idx_v, sem0)
    cp_r0 = pltpu.async_copy(tableT_hbm.at[wid], row0_v, sem0)
    cp_r1 = pltpu.async_copy(tableT_hbm.at[c1], row1_v, sem1)

    @pl.when(has2)
    def _():
        pltpu.async_copy(tableT_hbm.at[c1 + 1], row2_v, sem2)

    cp_idx.wait()
    cp_r0.wait()

    @plsc.parallel_loop(0, _BATCH, step=_L, unroll=8)
    def _(i):
        col0_v[pl.ds(i, _L)] = plsc.load_gather(row0_v, [idx_v[pl.ds(i, _L)]])

    wb0 = pltpu.async_copy(col0_v, outT_hbm.at[wid], semw)

    cp_r1.wait()

    @plsc.parallel_loop(r1 * _L, (r1 + n1) * _L, step=_L, unroll=8)
    def _(i):
        col2_v[pl.ds(i, _L)] = plsc.load_gather(row1_v, [idx_v[pl.ds(i, _L)]])

    @pl.when(has2)
    def _():
        pltpu.make_async_copy(tableT_hbm.at[c1 + 1], row2_v, sem2).wait()

        @plsc.parallel_loop(0, (_P2PW - n1) * _L, step=_L, unroll=8)
        def _(i):
            col2_v[pl.ds(i, _L)] = plsc.load_gather(row2_v,
                                                    [idx_v[pl.ds(i, _L)]])

    wbs = []
    for g in range(_NGRP):
        gg = q0 + g * _GRP
        c = _NW + gg // _CPC
        boff = (gg % _CPC) * _L
        wbs.append(pltpu.async_copy(
            col2_v.at[pl.ds(boff, _GRP * _L)],
            outT_hbm.at[c, pl.ds(boff, _GRP * _L)], semw))

    wb0.wait()
    for wb in wbs:
        wb.wait()


_freq_bias = functools.partial(
    pl.kernel,
    out_type=jax.ShapeDtypeStruct((_NUM_RELS, _BATCH), jnp.float32),
    mesh=plsc.VectorSubcoreMesh(core_axis_name="c", subcore_axis_name="s"),
    compiler_params=pltpu.CompilerParams(use_tc_tiling_on_sc=True,
                                         needs_layout_passes=False),
    scratch_types=[
        pltpu.VMEM((_BATCH,), jnp.int32),
        pltpu.VMEM((_NROWS,), jnp.float32),
        pltpu.VMEM((_NROWS,), jnp.float32),
        pltpu.VMEM((_NROWS,), jnp.float32),
        pltpu.VMEM((_BATCH,), jnp.float32),
        pltpu.VMEM((_BATCH,), jnp.float32),
        pltpu.SemaphoreType.DMA,
        pltpu.SemaphoreType.DMA,
        pltpu.SemaphoreType.DMA,
        pltpu.SemaphoreType.DMA,
    ],
)(_freq_bias_body)


def kernel(labels, obj_baseline_weight):
    labels = labels.astype(jnp.int32)
    pair_idx = labels[:, 0] * _NUM_OBJS + labels[:, 1]
    outT = _freq_bias(pair_idx, obj_baseline_weight.T)
    return outT.T


# speedup vs baseline: 1.0255x; 1.0255x over previous
import functools

import jax
import jax.numpy as jnp
from jax import lax
from jax.experimental import pallas as pl
from jax.experimental.pallas import tpu as pltpu
from jax.experimental.pallas import tpu_sc as plsc

_NUM_OBJS = 151
_NUM_RELS = 51
_BATCH = 16384
_NROWS = _NUM_OBJS * _NUM_OBJS

_NC = 2
_NS = 16
_NW = _NC * _NS
_L = 16

_SPLIT = _BATCH // 4


def _freq_bias_body(idx_hbm, tableT_hbm, outT_hbm,
                    idx_v, row0_v, row1_v, col0_v, col1_v,
                    sem0, sem1, semw):
    wid = lax.axis_index("s") * _NC + lax.axis_index("c")
    second = wid + _NW < _NUM_RELS

    cp_idx = pltpu.async_copy(idx_hbm, idx_v, sem0)
    cp_r0 = pltpu.async_copy(tableT_hbm.at[wid], row0_v, sem0)

    @pl.when(second)
    def _():
        pltpu.async_copy(tableT_hbm.at[wid + _NW], row1_v, sem1)

    cp_idx.wait()
    cp_r0.wait()

    @plsc.parallel_loop(0, _SPLIT, step=_L, unroll=8)
    def _(i):
        col0_v[pl.ds(i, _L)] = plsc.load_gather(row0_v, [idx_v[pl.ds(i, _L)]])

    @pl.when(second)
    def _():
        pltpu.make_async_copy(tableT_hbm.at[wid + _NW], row1_v, sem1).wait()

        @plsc.parallel_loop(_SPLIT, _BATCH, step=_L, unroll=8)
        def _(i):
            idx = idx_v[pl.ds(i, _L)]
            col0_v[pl.ds(i, _L)] = plsc.load_gather(row0_v, [idx])
            col1_v[pl.ds(i, _L)] = plsc.load_gather(row1_v, [idx])

        @plsc.parallel_loop(0, _SPLIT, step=_L, unroll=8)
        def _(i):
            col1_v[pl.ds(i, _L)] = plsc.load_gather(row1_v,
                                                    [idx_v[pl.ds(i, _L)]])

    @pl.when(jnp.logical_not(second))
    def _():
        @plsc.parallel_loop(_SPLIT, _BATCH, step=_L, unroll=8)
        def _(i):
            col0_v[pl.ds(i, _L)] = plsc.load_gather(row0_v,
                                                    [idx_v[pl.ds(i, _L)]])

    wb0 = pltpu.async_copy(col0_v, outT_hbm.at[wid], semw)

    @pl.when(second)
    def _():
        pltpu.async_copy(col1_v, outT_hbm.at[wid + _NW], semw).wait()

    wb0.wait()


_freq_bias = functools.partial(
    pl.kernel,
    out_type=jax.ShapeDtypeStruct((_NUM_RELS, _BATCH), jnp.float32),
    mesh=plsc.VectorSubcoreMesh(core_axis_name="c", subcore_axis_name="s"),
    compiler_params=pltpu.CompilerParams(use_tc_tiling_on_sc=True,
                                         needs_layout_passes=False),
    scratch_types=[
        pltpu.VMEM((_BATCH,), jnp.int32),
        pltpu.VMEM((_NROWS,), jnp.float32),
        pltpu.VMEM((_NROWS,), jnp.float32),
        pltpu.VMEM((_BATCH,), jnp.float32),
        pltpu.VMEM((_BATCH,), jnp.float32),
        pltpu.SemaphoreType.DMA,
        pltpu.SemaphoreType.DMA,
        pltpu.SemaphoreType.DMA,
    ],
)(_freq_bias_body)


def kernel(labels, obj_baseline_weight):
    labels = labels.astype(jnp.int32)
    pair_idx = labels[:, 0] * _NUM_OBJS + labels[:, 1]
    outT = _freq_bias(pair_idx, obj_baseline_weight.T)
    return outT.T
